# baseline (speedup 1.0000x reference)
import functools

import jax
import jax.numpy as jnp
from jax import lax
from jax.experimental import pallas as pl
from jax.experimental.pallas import tpu as pltpu
from jax.experimental.pallas import tpu_sc as plsc

NUM_REL = 500
EMB = 128
TD = 32
E = 128

_NW = 16
_RPW = E // _NW


def _sc_gather_body(sidx_hbm, nidx_hbm, emb_e_hbm,
                    src_rows_out, node_out,
                    idx_v, idx_n, rows_v, rows_n, sem_a, sem_n):
  wid = lax.axis_index("s")
  cp = pltpu.async_copy(sidx_hbm.at[wid], idx_v, sem_a)

  @pl.when(wid == 0)
  def _():
    pltpu.async_copy(nidx_hbm, idx_n, sem_n)

  cp.wait()
  g = pltpu.async_copy(emb_e_hbm.at[idx_v], rows_v, sem_a)

  @pl.when(wid == 0)
  def _():
    pltpu.make_async_copy(nidx_hbm, idx_n, sem_n).wait()
    pltpu.async_copy(emb_e_hbm.at[idx_n], rows_n, sem_n)

  g.wait()
  o = pltpu.async_copy(rows_v, src_rows_out.at[pl.ds(wid * _RPW, _RPW)], sem_a)

  @pl.when(wid == 0)
  def _():
    pltpu.make_async_copy(emb_e_hbm.at[idx_n], rows_n, sem_n).wait()
    pltpu.sync_copy(rows_n.at[pl.ds(0, 2)], node_out)

  o.wait()


def _make_sc_gather():
  return functools.partial(
      pl.kernel,
      out_type=[
          jax.ShapeDtypeStruct((E, EMB), jnp.float32),
          jax.ShapeDtypeStruct((2, EMB), jnp.float32),
      ],
      mesh=plsc.VectorSubcoreMesh(core_axis_name="c", subcore_axis_name="s",
                                  num_cores=1),
      scratch_types=[
          pltpu.VMEM((_RPW,), jnp.int32),
          pltpu.VMEM((8,), jnp.int32),
          pltpu.VMEM((_RPW, EMB), jnp.float32),
          pltpu.VMEM((8, EMB), jnp.float32),
          pltpu.SemaphoreType.DMA,
          pltpu.SemaphoreType.DMA,
      ],
  )(_sc_gather_body)


def _tc_body(pair_ref, ei_ref, et_ref, srcr_ref, node_ref, embr_ref, ts_ref,
             freq_ref, phase_ref, wm_ref, ws_ref, out_ref):
  pv = pair_ref[...]
  dsti = ei_ref[1:2, :]
  t_emb = jnp.cos(ts_ref[...] * freq_ref[...] + phase_ref[...])
  rel_onehot = (et_ref[...] == lax.broadcasted_iota(jnp.int32, (E, NUM_REL), 1)
                ).astype(jnp.float32)
  rel_rows = jnp.dot(rel_onehot, embr_ref[...],
                     preferred_element_type=jnp.float32)
  prod = srcr_ref[...] * rel_rows
  w1 = wm_ref[0:EMB, :]
  w2 = wm_ref[EMB:EMB + TD, :]

  def one_side(col):
    m = (dsti == pv[:, col:col + 1]).astype(jnp.float32)
    s_prod = jnp.dot(m, prod, preferred_element_type=jnp.float32)
    s_t = jnp.dot(m, t_emb, preferred_element_type=jnp.float32)
    inv_deg = 1.0 / jnp.maximum(jnp.sum(m), 1.0)
    agg = (jnp.dot(s_prod, w1, preferred_element_type=jnp.float32)
           + jnp.dot(s_t, w2, preferred_element_type=jnp.float32)
           ) * inv_deg
    self_t = jnp.dot(node_ref[col:col + 1, :], ws_ref[...],
                     preferred_element_type=jnp.float32)
    return jnp.maximum(agg + self_t, 0.0)

  out_ref[:, 0:EMB] = one_side(0)
  out_ref[:, EMB:2 * EMB] = one_side(1)


def kernel(one_pair, edge_index, edge_type, edge_ts, emb_e, emb_r, W_msg,
           W_self, t_freq, t_phase):
  ei = edge_index.astype(jnp.int32)
  etype = edge_type.astype(jnp.int32)
  pair = one_pair.astype(jnp.int32)
  sidx = ei[0].reshape(_NW, _RPW)
  nidx = jnp.concatenate([pair[0, :2], jnp.zeros((6,), jnp.int32)])

  src_rows, node = _make_sc_gather()(sidx, nidx, emb_e)

  out = pl.pallas_call(
      _tc_body,
      out_shape=jax.ShapeDtypeStruct((1, 2 * EMB), jnp.float32),
  )(pair, ei, etype.reshape(E, 1), src_rows, node, emb_r,
    edge_ts.astype(jnp.float32).reshape(E, 1), t_freq.reshape(1, TD),
    t_phase.reshape(1, TD), W_msg, W_self)
  return out
